# baseline (speedup 1.0000x reference)
import functools
import math

import jax
import jax.numpy as jnp
import numpy as np
from jax import lax
from jax.experimental import pallas as pl
from jax.experimental.pallas import tpu as pltpu
from jax.experimental.pallas import tpu_sc as plsc

_VOCAB = 1000000
_D = 64
_B = 1024
_L = 200
_N = _B * _L
_NC = 2
_NS = 16
_NW = _NC * _NS
_CHUNK = 128
_CPW = _N // (_NW * _CHUNK)
_CPL = _B // _CHUNK
_LANES = 16
_P = 5


def _make_pos_enc():
    pe = np.zeros((_L, _D), dtype=np.float32)
    position = np.arange(0.0, _L, dtype=np.float64)[:, None]
    div_term = np.exp(
        np.arange(0.0, _D, 2, dtype=np.float64) * -(math.log(10000.0) / _D))
    pe[:, 0::2] = np.sin(position * div_term).astype(np.float32)
    pe[:, 1::2] = np.cos(position * div_term).astype(np.float32)
    return pe


_PE = _make_pos_enc()

_mesh = plsc.VectorSubcoreMesh(
    core_axis_name="c", subcore_axis_name="s", num_cores=_NC, num_subcores=_NS)


@functools.partial(
    pl.kernel,
    out_type=jax.ShapeDtypeStruct((_B, _L, _D), jnp.float32),
    mesh=_mesh,
    compiler_params=pltpu.CompilerParams(use_tc_tiling_on_sc=False),
    scratch_types=[
        pltpu.VMEM((_CPW, _CHUNK), jnp.int32),
        pltpu.VMEM((_L, _D), jnp.float32),
        pltpu.VMEM((_P, _CHUNK, _D), jnp.float32),
        [pltpu.SemaphoreType.DMA] * _P,
        [pltpu.SemaphoreType.DMA] * _P,
        pltpu.SemaphoreType.DMA,
    ],
)
def _emb_pe_kernel(table_hbm, idx_hbm, pe_hbm, out_hbm,
                   idx_v, pe_v, rows_v, gsems, wsems, isem):
    wid = lax.axis_index("s") * _NC + lax.axis_index("c")
    chunk0 = wid * _CPW
    pltpu.async_copy(idx_hbm.at[wid], idx_v, isem)
    pltpu.sync_copy(pe_hbm, pe_v)
    pltpu.make_async_copy(idx_hbm.at[wid], idx_v, isem).wait()

    def gather_start(j, b):
        pltpu.async_copy(table_hbm.at[idx_v.at[j]], rows_v.at[b], gsems[b])

    for b in range(_P):
        gather_start(b, b)

    def outer(s, carry):
        for b in range(_P):
            j = s * _P + b
            g = chunk0 + j
            l = lax.div(g, _CPL)
            b0 = pl.multiple_of(lax.rem(g, _CPL) * _CHUNK, _CHUNK)
            pltpu.make_async_copy(
                table_hbm.at[idx_v.at[j]], rows_v.at[b], gsems[b]).wait()
            pes = [pe_v[l, pl.ds(k * _LANES, _LANES)]
                   for k in range(_D // _LANES)]

            def row_body(r, pes):
                for k in range(_D // _LANES):
                    sl = pl.ds(k * _LANES, _LANES)
                    rows_v[b, r, sl] = rows_v[b, r, sl] * 8.0 + pes[k]
                return pes

            lax.fori_loop(0, _CHUNK, row_body, tuple(pes), unroll=4)
            pltpu.async_copy(
                rows_v.at[b], out_hbm.at[pl.ds(b0, _CHUNK), l], wsems[b])

            @pl.when(s + 1 < _CPW // _P)
            def _():
                pltpu.make_async_copy(
                    rows_v.at[b], out_hbm.at[pl.ds(b0, _CHUNK), l],
                    wsems[b]).wait()
                gather_start(j + _P, b)

        return carry

    lax.fori_loop(0, _CPW // _P, outer, 0)
    for b in range(_P):
        j = _CPW - _P + b
        g = chunk0 + j
        l = lax.div(g, _CPL)
        b0 = pl.multiple_of(lax.rem(g, _CPL) * _CHUNK, _CHUNK)
        pltpu.make_async_copy(
            rows_v.at[b], out_hbm.at[pl.ds(b0, _CHUNK), l], wsems[b]).wait()


def kernel(x, table):
    xf = lax.bitcast_convert_type(x, jnp.float32)
    idxf = xf.T.reshape(_NW, _CPW, _CHUNK)
    idx3 = lax.bitcast_convert_type(idxf, jnp.int32)
    return _emb_pe_kernel(table, idx3, _PE)


# speedup vs baseline: 1.0003x; 1.0003x over previous
import functools
import math

import jax
import jax.numpy as jnp
import numpy as np
from jax import lax
from jax.experimental import pallas as pl
from jax.experimental.pallas import tpu as pltpu
from jax.experimental.pallas import tpu_sc as plsc

_VOCAB = 1000000
_D = 64
_B = 1024
_L = 200
_N = _B * _L
_NC = 2
_NS = 16
_NW = _NC * _NS
_CHUNK = 128
_CPW = _N // (_NW * _CHUNK)
_CPL = _B // _CHUNK
_LANES = 16
_P = 5


def _make_pos_enc():
    pe = np.zeros((_L, _D), dtype=np.float32)
    position = np.arange(0.0, _L, dtype=np.float64)[:, None]
    div_term = np.exp(
        np.arange(0.0, _D, 2, dtype=np.float64) * -(math.log(10000.0) / _D))
    pe[:, 0::2] = np.sin(position * div_term).astype(np.float32)
    pe[:, 1::2] = np.cos(position * div_term).astype(np.float32)
    return pe


_PE = _make_pos_enc()

_mesh = plsc.VectorSubcoreMesh(
    core_axis_name="c", subcore_axis_name="s", num_cores=_NC, num_subcores=_NS)


@functools.partial(
    pl.kernel,
    out_type=jax.ShapeDtypeStruct((_B, _L, _D), jnp.float32),
    mesh=_mesh,
    compiler_params=pltpu.CompilerParams(
        use_tc_tiling_on_sc=False, needs_layout_passes=False),
    scratch_types=[
        pltpu.VMEM((_CPW, _CHUNK), jnp.float32),
        pltpu.VMEM((_CPW, _CHUNK), jnp.int32),
        pltpu.VMEM((_L, _D), jnp.float32),
        pltpu.VMEM((_P, _CHUNK, _D), jnp.float32),
        [pltpu.SemaphoreType.DMA] * _P,
        [pltpu.SemaphoreType.DMA] * _P,
        pltpu.SemaphoreType.DMA,
    ],
)
def _emb_pe_kernel(table_hbm, idx_hbm, pe_hbm, out_hbm,
                   idxf_v, idx_v, pe_v, rows_v, gsems, wsems, isem):
    wid = lax.axis_index("s") * _NC + lax.axis_index("c")
    chunk0 = wid * _CPW
    pltpu.async_copy(idx_hbm.at[wid], idxf_v, isem)
    pltpu.sync_copy(pe_hbm, pe_v)
    pltpu.make_async_copy(idx_hbm.at[wid], idxf_v, isem).wait()

    def idx_cast(j, carry):
        for k in range(_CHUNK // _LANES):
            sl = pl.ds(k * _LANES, _LANES)
            idx_v[j, sl] = plsc.bitcast(idxf_v[j, sl], jnp.int32)
        return carry

    lax.fori_loop(0, _CPW, idx_cast, 0)

    def gather_start(j, b):
        pltpu.async_copy(table_hbm.at[idx_v.at[j]], rows_v.at[b], gsems[b])

    for b in range(_P):
        gather_start(b, b)

    def outer(s, carry):
        for b in range(_P):
            j = s * _P + b
            g = chunk0 + j
            l = lax.div(g, _CPL)
            b0 = pl.multiple_of(lax.rem(g, _CPL) * _CHUNK, _CHUNK)
            pltpu.make_async_copy(
                table_hbm.at[idx_v.at[j]], rows_v.at[b], gsems[b]).wait()
            pes = [pe_v[l, pl.ds(k * _LANES, _LANES)]
                   for k in range(_D // _LANES)]

            def row_body(r, pes):
                for k in range(_D // _LANES):
                    sl = pl.ds(k * _LANES, _LANES)
                    rows_v[b, r, sl] = rows_v[b, r, sl] * 8.0 + pes[k]
                return pes

            lax.fori_loop(0, _CHUNK, row_body, tuple(pes), unroll=4)
            pltpu.async_copy(
                rows_v.at[b], out_hbm.at[pl.ds(b0, _CHUNK), l], wsems[b])

            @pl.when(s + 1 < _CPW // _P)
            def _():
                pltpu.make_async_copy(
                    rows_v.at[b], out_hbm.at[pl.ds(b0, _CHUNK), l],
                    wsems[b]).wait()
                gather_start(j + _P, b)

        return carry

    lax.fori_loop(0, _CPW // _P, outer, 0)
    for b in range(_P):
        j = _CPW - _P + b
        g = chunk0 + j
        l = lax.div(g, _CPL)
        b0 = pl.multiple_of(lax.rem(g, _CPL) * _CHUNK, _CHUNK)
        pltpu.make_async_copy(
            rows_v.at[b], out_hbm.at[pl.ds(b0, _CHUNK), l], wsems[b]).wait()


def kernel(x, table):
    xf = lax.bitcast_convert_type(x, jnp.float32)
    idxf = xf.T.reshape(_NW, _CPW, _CHUNK)
    return _emb_pe_kernel(table, idxf, _PE)


# speedup vs baseline: 1.0010x; 1.0007x over previous
import functools
import math

import jax
import jax.numpy as jnp
import numpy as np
from jax import lax
from jax.experimental import pallas as pl
from jax.experimental.pallas import tpu as pltpu
from jax.experimental.pallas import tpu_sc as plsc

_VOCAB = 1000000
_D = 64
_B = 1024
_L = 200
_N = _B * _L
_NC = 2
_NS = 16
_NW = _NC * _NS
_CHUNK = 128
_CPW = _N // (_NW * _CHUNK)
_CPL = _B // _CHUNK
_LANES = 16
_P = 5


def _make_pos_enc():
    pe = np.zeros((_L, _D), dtype=np.float32)
    position = np.arange(0.0, _L, dtype=np.float64)[:, None]
    div_term = np.exp(
        np.arange(0.0, _D, 2, dtype=np.float64) * -(math.log(10000.0) / _D))
    pe[:, 0::2] = np.sin(position * div_term).astype(np.float32)
    pe[:, 1::2] = np.cos(position * div_term).astype(np.float32)
    return pe


_PE = _make_pos_enc()

_mesh = plsc.VectorSubcoreMesh(
    core_axis_name="c", subcore_axis_name="s", num_cores=_NC, num_subcores=_NS)


@functools.partial(
    pl.kernel,
    out_type=jax.ShapeDtypeStruct((_B, _L, _D), jnp.float32),
    mesh=_mesh,
    compiler_params=pltpu.CompilerParams(
        use_tc_tiling_on_sc=False, needs_layout_passes=False),
    scratch_types=[
        pltpu.VMEM((_CPW, _CHUNK), jnp.float32),
        pltpu.VMEM((_CPW, _CHUNK), jnp.int32),
        pltpu.VMEM((_L, _D), jnp.float32),
        pltpu.VMEM((_P, _CHUNK, _D), jnp.float32),
        [pltpu.SemaphoreType.DMA] * _P,
        [pltpu.SemaphoreType.DMA] * _P,
        pltpu.SemaphoreType.DMA,
    ],
)
def _emb_pe_kernel(table_hbm, idx_hbm, pe_hbm, out_hbm,
                   idxf_v, idx_v, pe_v, rows_v, gsems, wsems, isem):
    wid = lax.axis_index("s") * _NC + lax.axis_index("c")
    chunk0 = wid * _CPW

    def idx_fetch(j, carry):
        g = chunk0 + j
        l = lax.div(g, _CPL)
        pltpu.async_copy(
            idx_hbm.at[lax.div(l, 8), lax.rem(g, _CPL), lax.rem(l, 8)],
            idxf_v.at[j], isem)
        return carry

    lax.fori_loop(0, _CPW, idx_fetch, 0)
    pltpu.sync_copy(pe_hbm, pe_v)

    def idx_drain(j, carry):
        pltpu.make_async_copy(idx_hbm.at[0, 0, 0], idxf_v.at[0], isem).wait()
        return carry

    lax.fori_loop(0, _CPW, idx_drain, 0)

    def idx_cast(j, carry):
        for k in range(_CHUNK // _LANES):
            sl = pl.ds(k * _LANES, _LANES)
            idx_v[j, sl] = plsc.bitcast(idxf_v[j, sl], jnp.int32)
        return carry

    lax.fori_loop(0, _CPW, idx_cast, 0)

    def gather_start(j, b):
        pltpu.async_copy(table_hbm.at[idx_v.at[j]], rows_v.at[b], gsems[b])

    for b in range(_P):
        gather_start(b, b)

    def outer(s, carry):
        for b in range(_P):
            j = s * _P + b
            g = chunk0 + j
            l = lax.div(g, _CPL)
            b0 = pl.multiple_of(lax.rem(g, _CPL) * _CHUNK, _CHUNK)
            pltpu.make_async_copy(
                table_hbm.at[idx_v.at[j]], rows_v.at[b], gsems[b]).wait()
            pes = [pe_v[l, pl.ds(k * _LANES, _LANES)]
                   for k in range(_D // _LANES)]

            def row_body(r, pes):
                for k in range(_D // _LANES):
                    sl = pl.ds(k * _LANES, _LANES)
                    rows_v[b, r, sl] = rows_v[b, r, sl] * 8.0 + pes[k]
                return pes

            lax.fori_loop(0, _CHUNK, row_body, tuple(pes), unroll=4)
            pltpu.async_copy(
                rows_v.at[b], out_hbm.at[pl.ds(b0, _CHUNK), l], wsems[b])

            @pl.when(s + 1 < _CPW // _P)
            def _():
                pltpu.make_async_copy(
                    rows_v.at[b], out_hbm.at[pl.ds(b0, _CHUNK), l],
                    wsems[b]).wait()
                gather_start(j + _P, b)

        return carry

    lax.fori_loop(0, _CPW // _P, outer, 0)
    for b in range(_P):
        j = _CPW - _P + b
        g = chunk0 + j
        l = lax.div(g, _CPL)
        b0 = pl.multiple_of(lax.rem(g, _CPL) * _CHUNK, _CHUNK)
        pltpu.make_async_copy(
            rows_v.at[b], out_hbm.at[pl.ds(b0, _CHUNK), l], wsems[b]).wait()


def kernel(x, table):
    xf = lax.bitcast_convert_type(x, jnp.float32)
    idx4f = xf.reshape(_CPL, _CHUNK, _L // 8, 8).transpose(2, 0, 3, 1)
    return _emb_pe_kernel(table, idx4f, _PE)
